# baseline (speedup 1.0000x reference)
import jax
import jax.numpy as jnp
from jax import lax
from jax.experimental import pallas as pl
from jax.experimental.pallas import tpu as pltpu
from jax.experimental.pallas import tpu_sc as plsc

_C = 384
_NW = 32
_NS = 16


def _sc_body(mix_hbm, ref_hbm, out_val_hbm, out_idx_hbm,
             spmem, rbufs, mbufs, vbufs, ibufs, rsems, csems, msems, osems):
    B, A, N, L = ref_hbm.shape
    lsc = (L // 128) * 128
    mix_pitch = L
    nchunk = lsc // _C
    ntasks = B * A * nchunk
    kmax = (ntasks + _NW - 1) // _NW
    s_id = lax.axis_index("s")
    w = s_id * 2 + lax.axis_index("c")

    def task_coords(t):
        ba = t // nchunk
        chunk = t - ba * nchunk
        b = ba // A
        a = ba - b * A
        c0 = chunk * _C
        return b, a, c0

    def start_stage_a(t, i):
        b, a, c0 = task_coords(t)
        pltpu.make_async_copy(
            ref_hbm.at[b, a, :, pl.ds(c0, _C)], spmem.at[s_id, i],
            rsems[i]).start()

    def wait_stage_a(i):
        pltpu.make_async_copy(
            ref_hbm.at[0, 0, :, pl.ds(0, _C)], spmem.at[0, i],
            rsems[i]).wait()

    def start_stage_b(t, i):
        b, _, c0 = task_coords(t)
        pltpu.make_async_copy(spmem.at[s_id, i], rbufs[i], csems[i]).start()
        pltpu.make_async_copy(
            mix_hbm.at[pl.ds(b * mix_pitch + c0, _C)], mbufs[i],
            msems[i]).start()

    def wait_stage_b(i):
        pltpu.make_async_copy(spmem.at[0, i], rbufs[i], csems[i]).wait()
        pltpu.make_async_copy(
            mix_hbm.at[pl.ds(0, _C)], mbufs[i], msems[i]).wait()

    def start_out(t, i):
        b, a, c0 = task_coords(t)
        pltpu.make_async_copy(
            vbufs[i], out_val_hbm.at[b, a, 0, pl.ds(c0, _C)],
            osems[i]).start()
        pltpu.make_async_copy(
            ibufs[i], out_idx_hbm.at[b, a, 0, pl.ds(c0, _C)],
            osems[i]).start()

    def wait_out(i):
        pltpu.make_async_copy(
            vbufs[i], out_val_hbm.at[0, 0, 0, pl.ds(0, _C)], osems[i]).wait()
        pltpu.make_async_copy(
            ibufs[i], out_idx_hbm.at[0, 0, 0, pl.ds(0, _C)], osems[i]).wait()

    def compute(i):
        rbuf, mbuf, vbuf, ibuf = rbufs[i], mbufs[i], vbufs[i], ibufs[i]

        def col(j, carry):
            for u in range(2):
                s = (2 * j + u) * 16
                m = mbuf[pl.ds(s, 16)]
                best = m * rbuf[0, pl.ds(s, 16)]
                idx = jnp.zeros((16,), jnp.int32)
                for n in range(1, N):
                    q = m * rbuf[n, pl.ds(s, 16)]
                    gt = q > best
                    idx = jnp.where(gt, jnp.full((16,), n, jnp.int32), idx)
                    best = jnp.maximum(q, best)
                vbuf[pl.ds(s, 16)] = best
                ibuf[pl.ds(s, 16)] = idx
            return carry

        lax.fori_loop(0, _C // 32, col, 0)

    start_stage_a(w, 0)

    @pl.when(w + _NW < ntasks)
    def _():
        start_stage_a(w + _NW, 1)

    wait_stage_a(0)
    start_stage_b(w, 0)

    def outer(o, carry):
        for phase in range(2):
            k = 2 * o + phase
            t = w + k * _NW

            @pl.when(t < ntasks)
            def _():
                wait_stage_b(phase)

                @pl.when(t + 2 * _NW < ntasks)
                def _():
                    start_stage_a(t + 2 * _NW, phase)

                @pl.when(t + _NW < ntasks)
                def _():
                    wait_stage_a(1 - phase)
                    start_stage_b(t + _NW, 1 - phase)

                @pl.when(o >= 1)
                def _():
                    wait_out(phase)

                compute(phase)
                start_out(t, phase)

        return carry

    lax.fori_loop(0, (kmax + 1) // 2, outer, 0)

    wait_out(0)
    wait_out(1)


def _sc_call(mix_flat, ref_panel):
    B, A, N, L = ref_panel.shape
    lsc = (L // 128) * 128
    mesh = plsc.VectorSubcoreMesh(core_axis_name="c", subcore_axis_name="s")
    out_type = (
        jax.ShapeDtypeStruct((B, A, 1, lsc), jnp.float32),
        jax.ShapeDtypeStruct((B, A, 1, lsc), jnp.int32),
    )
    scratch = [
        pltpu.VMEM_SHARED((_NS, 2, N, _C), jnp.float32),
        [pltpu.VMEM((N, _C), jnp.float32)] * 2,
        [pltpu.VMEM((_C,), jnp.float32)] * 2,
        [pltpu.VMEM((_C,), jnp.float32)] * 2,
        [pltpu.VMEM((_C,), jnp.int32)] * 2,
        [pltpu.SemaphoreType.DMA] * 2,
        [pltpu.SemaphoreType.DMA] * 2,
        [pltpu.SemaphoreType.DMA] * 2,
        [pltpu.SemaphoreType.DMA] * 2,
    ]
    f = pl.kernel(
        _sc_body,
        out_type=out_type,
        mesh=mesh,
        scratch_types=scratch,
    )
    return f(mix_flat, ref_panel)


def _tc_tail_body(mix_ref, ref_ref, val_ref, idx_ref):
    n = ref_ref.shape[2]
    r = ref_ref[0, 0]
    m = mix_ref[0]
    prod = m * r
    maxv = jnp.max(prod, axis=0, keepdims=True)
    iota = lax.broadcasted_iota(jnp.int32, prod.shape, 0)
    cand = jnp.where(prod == maxv, iota, n)
    idx = jnp.min(cand, axis=0, keepdims=True)
    val_ref[0, 0] = maxv
    idx_ref[0, 0] = idx


def _tc_tail_call(input_mixed, ref_panel):
    B, A, N, L = ref_panel.shape
    lsc = (L // 128) * 128
    jtail = lsc // 128
    ltail = L - lsc
    mix3 = input_mixed.reshape(B, 1, L)
    out_shape = (
        jax.ShapeDtypeStruct((B, A, 1, ltail), jnp.float32),
        jax.ShapeDtypeStruct((B, A, 1, ltail), jnp.int32),
    )
    return pl.pallas_call(
        _tc_tail_body,
        grid=(B, A),
        in_specs=[
            pl.BlockSpec((1, 1, 128), lambda b, a: (b, 0, jtail)),
            pl.BlockSpec((1, 1, N, 128), lambda b, a: (b, a, 0, jtail)),
        ],
        out_specs=[
            pl.BlockSpec((1, 1, 1, 128), lambda b, a: (b, a, 0, 0)),
            pl.BlockSpec((1, 1, 1, 128), lambda b, a: (b, a, 0, 0)),
        ],
        out_shape=out_shape,
    )(mix3, ref_panel)


def kernel(input_mixed, ref_panel):
    B, A, N, L = ref_panel.shape
    lsc = (L // 128) * 128
    mix_flat = input_mixed.reshape(-1)
    sc_val, sc_idx = _sc_call(mix_flat, ref_panel)
    tail_val, tail_idx = _tc_tail_call(input_mixed, ref_panel)
    pooled = jnp.concatenate([sc_val, tail_val], axis=3)
    indices = jnp.concatenate([sc_idx[:, :, 0, :], tail_idx[:, :, 0, :]],
                              axis=2)
    return pooled, indices
